# baseline (speedup 1.0000x reference)
import jax
import jax.numpy as jnp
from jax.experimental import pallas as pl
from jax.experimental.pallas import tpu as pltpu

_B = 512
_F = 4 * 64 * 64
_BB = 8


def _body(t_ref, sa_ref, som_ref, x_ref, n_ref, o_ref):
    i = pl.program_id(0)
    base = i * _BB
    for k in range(_BB):
        tk = t_ref[base + k]
        a = sa_ref[tk]
        b = som_ref[tk]
        o_ref[k, :] = a * x_ref[k, :] + b * n_ref[k, :]


def kernel(x0, t, noise, sqrt_alphas_cumprod, sqrt_one_minus_alphas_cumprod):
    x2 = x0.reshape(_B, _F)
    n2 = noise.reshape(_B, _F)
    grid_spec = pltpu.PrefetchScalarGridSpec(
        num_scalar_prefetch=3,
        grid=(_B // _BB,),
        in_specs=[
            pl.BlockSpec((_BB, _F), lambda i, *_: (i, 0)),
            pl.BlockSpec((_BB, _F), lambda i, *_: (i, 0)),
        ],
        out_specs=pl.BlockSpec((_BB, _F), lambda i, *_: (i, 0)),
    )
    out = pl.pallas_call(
        _body,
        grid_spec=grid_spec,
        out_shape=jax.ShapeDtypeStruct((_B, _F), jnp.float32),
    )(
        t.astype(jnp.int32),
        sqrt_alphas_cumprod,
        sqrt_one_minus_alphas_cumprod,
        x2,
        n2,
    )
    return out.reshape(x0.shape)


# speedup vs baseline: 1.2022x; 1.2022x over previous
import jax
import jax.numpy as jnp
from jax.experimental import pallas as pl
from jax.experimental.pallas import tpu as pltpu

_B = 512
_F = 4 * 64 * 64
_BB = 32


def _body(t_ref, sa_ref, som_ref, x_ref, n_ref, o_ref):
    i = pl.program_id(0)
    base = i * _BB
    a_list = []
    b_list = []
    for k in range(_BB):
        tk = t_ref[base + k]
        a_list.append(sa_ref[tk])
        b_list.append(som_ref[tk])
    a_col = jnp.stack(a_list).reshape(_BB, 1)
    b_col = jnp.stack(b_list).reshape(_BB, 1)
    o_ref[:, :] = a_col * x_ref[:, :] + b_col * n_ref[:, :]


def kernel(x0, t, noise, sqrt_alphas_cumprod, sqrt_one_minus_alphas_cumprod):
    x2 = x0.reshape(_B, _F)
    n2 = noise.reshape(_B, _F)
    grid_spec = pltpu.PrefetchScalarGridSpec(
        num_scalar_prefetch=3,
        grid=(_B // _BB,),
        in_specs=[
            pl.BlockSpec((_BB, _F), lambda i, *_: (i, 0)),
            pl.BlockSpec((_BB, _F), lambda i, *_: (i, 0)),
        ],
        out_specs=pl.BlockSpec((_BB, _F), lambda i, *_: (i, 0)),
    )
    out = pl.pallas_call(
        _body,
        grid_spec=grid_spec,
        out_shape=jax.ShapeDtypeStruct((_B, _F), jnp.float32),
    )(
        t.astype(jnp.int32),
        sqrt_alphas_cumprod,
        sqrt_one_minus_alphas_cumprod,
        x2,
        n2,
    )
    return out.reshape(x0.shape)
